# baseline (speedup 1.0000x reference)
import dataclasses
import functools
import math

import jax
import jax.numpy as jnp
import numpy as np
from jax import lax
from jax.experimental import pallas as pl
from jax.experimental.pallas import tpu as pltpu
from jax.experimental.pallas import tpu_sc as plsc

NP_ = 10000
KCAP = 64
FILTER = 4
EXTENT = np.float32(4 * 6 * 0.025)
RADIUS = np.float32(EXTENT / 2.0)
R2 = np.float32(RADIUS * RADIUS)
BOX = np.float32(3.3)
G = 11
INVH = np.float32(G / BOX)
NCELL = G * G * G

NC, NS = 2, 16
NW = NC * NS
QCH = 320
NPAD = NW * QCH
QB = 32
NSTARTS = 1336
RB = 512

_ATAN_C = (0.9999994161532382, -0.33330223018999183, 0.19951119254100916,
           -0.13933275185445243, 0.09709477935281635, -0.05688276598933052,
           0.02256821902867305, -0.004257820308274221)
_HALF_PI = np.float32(math.pi / 2)
_FOUR_OVER_PI = np.float32(4.0 / math.pi)


def _atan(t):
    at = jnp.abs(t)
    inv = at > 1.0
    z = jnp.where(inv, 1.0 / jnp.where(inv, t, 1.0), t)
    z2 = z * z
    p = jnp.float32(_ATAN_C[-1])
    for c in _ATAN_C[-2::-1]:
        p = p * z2 + np.float32(c)
    p = p * z
    return jnp.where(inv, jnp.sign(t) * _HALF_PI - p, p)


def _sc_body(qch, qpos_hbm, psx_hbm, psy_hbm, psz_hbm, vsx_hbm, vsy_hbm,
             vsz_hbm, starts_hbm, out_hbm,
             psx_v, psy_v, psz_v, vsx_v, vsy_v, vsz_v, starts_v,
             qpos_v, stage_v):
    wid = lax.axis_index("s") * NC + lax.axis_index("c")
    qbase = wid * qch
    pltpu.sync_copy(qpos_hbm.at[pl.ds(qbase * 3, qch * 3)], qpos_v)
    pltpu.sync_copy(psx_hbm, psx_v)
    pltpu.sync_copy(psy_hbm, psy_v)
    pltpu.sync_copy(psz_hbm, psz_v)
    pltpu.sync_copy(vsx_hbm, vsx_v)
    pltpu.sync_copy(vsy_hbm, vsy_v)
    pltpu.sync_copy(vsz_hbm, vsz_v)
    pltpu.sync_copy(starts_hbm, starts_v)

    lanes = lax.iota(jnp.int32, 16)
    big = jnp.full((16,), 1e9, jnp.float32)

    @pl.loop(0, qch, step=QB)
    def _qblock(qb):
        def _q(ql):
            qi = qb + ql
            qv = plsc.load_gather(qpos_v, [qi * 3 + jnp.minimum(lanes, 2)])
            qx = qv[0]
            qy = qv[1]
            qz = qv[2]
            def _ifloor(v):
                c = v.astype(jnp.int32)
                return jnp.where(c.astype(jnp.float32) > v, c - 1, c)

            cx = _ifloor(qx * INVH)
            cy = _ifloor(qy * INVH)
            cz = _ifloor(qz * INVH)
            qlv = jnp.full((16,), ql, jnp.int32)
            row6 = jnp.full((16,), 6, jnp.int32)
            for c in range(4):
                plsc.store_scatter(stage_v, [qlv, row6, lanes + 16 * c], big)
            z0 = jnp.maximum(cz - 1, 0)
            z1 = jnp.minimum(cz + 1, G - 1)

            def scan_run(cnt, dxc, dyc):
                gx = cx + dxc
                gy = cy + dyc
                ok = (gx >= 0) & (gx < G) & (gy >= 0) & (gy < G)
                col = (gx * G + gy) * G
                col = jnp.where(ok, col, 0)
                se_idx = jnp.where(lanes == 0, col + z0, col + z1 + 1)
                sev = plsc.load_gather(starts_v, [se_idx])
                s = jnp.where(ok, sev[0], 0)
                e = jnp.where(ok, sev[1], 0)
                nch = (e - s + 15) // 16

                def chunk(j, cnt):
                    base = s + j * 16
                    idxv = base + lanes
                    m_in = idxv < e
                    idxc = jnp.where(m_in, idxv, 0)
                    px = plsc.load_gather(psx_v, [idxc])
                    py = plsc.load_gather(psy_v, [idxc])
                    pz = plsc.load_gather(psz_v, [idxc])
                    dx = px - qx
                    dy = py - qy
                    dz = pz - qz
                    d2 = dx * dx + dy * dy + dz * dz
                    hit = m_in & (d2 <= R2)
                    hi = hit.astype(jnp.int32)
                    cpos = cnt + plsc.cumsum(hi) - 1
                    new_cnt = cpos[15] + 1

                    @pl.when(new_cnt > cnt)
                    def _store():
                        okm = hit & (cpos < KCAP)
                        vx = plsc.load_gather(vsx_v, [idxc])
                        vy = plsc.load_gather(vsy_v, [idxc])
                        vz = plsc.load_gather(vsz_v, [idxc])
                        r0 = jnp.zeros((16,), jnp.int32)
                        plsc.store_scatter(stage_v, [qlv, r0, cpos], dx, mask=okm)
                        plsc.store_scatter(stage_v, [qlv, r0 + 1, cpos], dy, mask=okm)
                        plsc.store_scatter(stage_v, [qlv, r0 + 2, cpos], dz, mask=okm)
                        plsc.store_scatter(stage_v, [qlv, r0 + 3, cpos], vx, mask=okm)
                        plsc.store_scatter(stage_v, [qlv, r0 + 4, cpos], vy, mask=okm)
                        plsc.store_scatter(stage_v, [qlv, r0 + 5, cpos], vz, mask=okm)
                        plsc.store_scatter(stage_v, [qlv, row6, cpos], d2, mask=okm)

                    return new_cnt

                return lax.fori_loop(0, nch, chunk, cnt)

            cnt = jnp.int32(0)
            for dxc in (-1, 0, 1):
                for dyc in (-1, 0, 1):
                    cnt = scan_run(cnt, dxc, dyc)

        @pl.loop(0, QB)
        def _qloop(ql):
            _q(ql)

        pltpu.sync_copy(stage_v, out_hbm.at[pl.ds(qbase + qb, QB)])


@functools.cache
def _sc_neighbors_kernel(qch):
  cp = pltpu.CompilerParams()
  if "needs_layout_passes" in pltpu.CompilerParams.__dataclass_fields__:
    cp = dataclasses.replace(cp, needs_layout_passes=False)
  return pl.kernel(
    functools.partial(_sc_body, qch),
    out_type=jax.ShapeDtypeStruct((NW * qch, 8, KCAP), jnp.float32),
    mesh=plsc.VectorSubcoreMesh(core_axis_name="c", subcore_axis_name="s",
                                num_cores=NC, num_subcores=NS),
    scratch_types=[
        pltpu.VMEM((NPAD,), jnp.float32),
        pltpu.VMEM((NPAD,), jnp.float32),
        pltpu.VMEM((NPAD,), jnp.float32),
        pltpu.VMEM((NPAD,), jnp.float32),
        pltpu.VMEM((NPAD,), jnp.float32),
        pltpu.VMEM((NPAD,), jnp.float32),
        pltpu.VMEM((NSTARTS,), jnp.int32),
        pltpu.VMEM((qch * 3,), jnp.float32),
        pltpu.VMEM((QB, 8, KCAP), jnp.float32),
    ],
    compiler_params=cp,
  )


def _tc_body(e_ref, w_ref, o_ref):
    e = e_ref[...]

    def plane(r):
        return jnp.transpose(e[:, r, :], (1, 0))

    d2 = plane(6)
    valid = d2 <= R2
    fv = valid.astype(jnp.float32)

    def san(r, fill):
        return jnp.where(valid, plane(r), np.float32(fill))

    dx = san(0, 1.0)
    dy = san(1, 0.0)
    dz = san(2, 0.0)
    vx = san(3, 0.0)
    vy = san(4, 0.0)
    vz = san(5, 0.0)

    scale = np.float32(2.0 / EXTENT)
    x = dx * scale
    y = dy * scale
    z = dz * scale

    sq = x * x + y * y + z * z
    norm = jnp.sqrt(jnp.maximum(sq, 1e-20))
    xy_sq = x * x + y * y
    cond = (5.0 * z * z / 4.0) <= xy_sq
    s_a = norm / jnp.sqrt(jnp.maximum(xy_sq, 1e-20))
    xa, ya, za = x * s_a, y * s_a, 1.5 * z
    s_b = jnp.sqrt(3.0 * norm / jnp.maximum(norm + jnp.abs(z), 1e-20))
    xb, yb, zb = x * s_b, y * s_b, jnp.sign(z) * norm
    xo = jnp.where(cond, xa, xb)
    yo = jnp.where(cond, ya, yb)
    zo = jnp.where(cond, za, zb)
    zero = sq < 1e-20
    x = jnp.where(zero, x, xo)
    y = jnp.where(zero, y, yo)
    z = jnp.where(zero, z, zo)

    sq_xy = x * x + y * y
    norm_xy = jnp.sqrt(jnp.maximum(sq_xy, 1e-20))
    cond = jnp.abs(y) <= jnp.abs(x)
    safe_x = jnp.where(jnp.abs(x) > 1e-10, x, np.float32(1e-10))
    safe_y = jnp.where(jnp.abs(y) > 1e-10, y, np.float32(1e-10))
    xa = jnp.sign(x) * norm_xy
    ya = jnp.sign(x) * _FOUR_OVER_PI * norm_xy * _atan(y / safe_x)
    yb = jnp.sign(y) * norm_xy
    xb = jnp.sign(y) * _FOUR_OVER_PI * norm_xy * _atan(x / safe_y)
    xo = jnp.where(cond, xa, xb)
    yo = jnp.where(cond, ya, yb)
    zero = sq_xy < 1e-20
    x = jnp.where(zero, x, xo)
    y = jnp.where(zero, y, yo)

    fx = (x * 0.5 + 0.5) * (FILTER - 1)
    fy = (y * 0.5 + 0.5) * (FILTER - 1)
    fz = (z * 0.5 + 0.5) * (FILTER - 1)

    def axw(f):
        f0 = jnp.floor(f)
        frac = f - f0
        i0 = jnp.clip(f0.astype(jnp.int32), 0, FILTER - 1)
        i1 = jnp.clip(f0.astype(jnp.int32) + 1, 0, FILTER - 1)
        return [(1.0 - frac) * (i0 == a).astype(jnp.float32)
                + frac * (i1 == a).astype(jnp.float32) for a in range(FILTER)]

    r_sqr = d2 / R2
    w_win = jnp.clip((1.0 - r_sqr) ** 3, 0.0, 1.0)
    imp = w_win * fv

    axl = axw(fx)
    ayl = axw(fy)
    azl = axw(fz)
    fim = [imp * vx, imp * vy, imp * vz]
    gp = [[axl[xx] * fim[i] for i in range(3)] for xx in range(4)]

    outs = [jnp.zeros((RB,), jnp.float32) for _ in range(3)]
    for a in range(4):
        for b in range(4):
            azay = azl[a] * ayl[b]
            for xx in range(4):
                for i in range(3):
                    s = jnp.sum(azay * gp[xx][i], axis=0)
                    widx = ((a * 4 + b) * 4 + xx) * 3 + i
                    for o in range(3):
                        outs[o] = outs[o] + s * w_ref[widx, o]
    o_ref[...] = jnp.stack(outs, axis=0)


@functools.cache
def _tc_conv(npart):
  return pl.pallas_call(
    _tc_body,
    grid=(npart // RB,),
    in_specs=[
        pl.BlockSpec((RB, 8, KCAP), lambda i: (i, 0, 0)),
        pl.BlockSpec(memory_space=pltpu.SMEM),
    ],
    out_specs=pl.BlockSpec((3, RB), lambda i: (0, i)),
    out_shape=jax.ShapeDtypeStruct((3, npart), jnp.float32),
  )


def kernel(pos, vel, W):
    pos = pos.astype(jnp.float32)
    vel = vel.astype(jnp.float32)
    cxyz = jnp.clip((pos * INVH).astype(jnp.int32), 0, G - 1)
    cid = (cxyz[:, 0] * G + cxyz[:, 1]) * G + cxyz[:, 2]
    sidx = jnp.argsort(cid)
    counts = jnp.zeros((NCELL,), jnp.int32).at[cid].add(1)
    starts = jnp.concatenate(
        [jnp.zeros((1,), jnp.int32), jnp.cumsum(counts, dtype=jnp.int32)])
    starts = jnp.pad(starts, (0, NSTARTS - (NCELL + 1)), constant_values=NP_)
    ps = pos[sidx]
    vs = vel[sidx]
    padn = NPAD - NP_

    def pad1(a):
        return jnp.pad(a, (0, padn))

    qpos = jnp.pad(pos, ((0, padn), (0, 0))).reshape(-1)
    planes = (pad1(ps[:, 0]), pad1(ps[:, 1]), pad1(ps[:, 2]),
              pad1(vs[:, 0]), pad1(vs[:, 1]), pad1(vs[:, 2]), starts)
    wmat = W.astype(jnp.float32).reshape(FILTER ** 3 * 3, 3)
    half = NPAD // 2
    sc = _sc_neighbors_kernel(half // NW)
    e1 = sc(qpos[:half * 3], *planes)
    e2 = sc(qpos[half * 3:], *planes)
    o1 = _tc_conv(half)(e1, wmat)
    o2 = _tc_conv(half)(e2, wmat)
    out = jnp.concatenate([o1, o2], axis=1)
    wc = 0.125 * jnp.sum(W.astype(jnp.float32)[1:3, 1:3, 1:3], axis=(0, 1, 2))
    return out[:, :NP_].T - vel @ wc


# speedup vs baseline: 1.0947x; 1.0947x over previous
import dataclasses
import functools
import math

import jax
import jax.numpy as jnp
import numpy as np
from jax import lax
from jax.experimental import pallas as pl
from jax.experimental.pallas import tpu as pltpu
from jax.experimental.pallas import tpu_sc as plsc

NP_ = 10000
KCAP = 64
FILTER = 4
EXTENT = np.float32(4 * 6 * 0.025)
RADIUS = np.float32(EXTENT / 2.0)
R2 = np.float32(RADIUS * RADIUS)
BOX = np.float32(3.3)
G = 11
INVH = np.float32(G / BOX)
NCELL = G * G * G

NC, NS = 2, 16
NW = NC * NS
QCH = 320
NPAD = NW * QCH
QB = 16
NSTARTS = 1336
RB = 512

_ATAN_C = (0.9999994161532382, -0.33330223018999183, 0.19951119254100916,
           -0.13933275185445243, 0.09709477935281635, -0.05688276598933052,
           0.02256821902867305, -0.004257820308274221)
_HALF_PI = np.float32(math.pi / 2)
_FOUR_OVER_PI = np.float32(4.0 / math.pi)


def _atan(t):
    at = jnp.abs(t)
    inv = at > 1.0
    z = jnp.where(inv, 1.0 / jnp.where(inv, t, 1.0), t)
    z2 = z * z
    p = jnp.float32(_ATAN_C[-1])
    for c in _ATAN_C[-2::-1]:
        p = p * z2 + np.float32(c)
    p = p * z
    return jnp.where(inv, jnp.sign(t) * _HALF_PI - p, p)


def _sc_body(qch, qpos_hbm, psx_hbm, psy_hbm, psz_hbm, vsx_hbm, vsy_hbm,
             vsz_hbm, starts_hbm, out_hbm,
             psx_v, psy_v, psz_v, vsx_v, vsy_v, vsz_v, starts_v,
             qpos_v, stage_v):
    wid = lax.axis_index("s") * NC + lax.axis_index("c")
    qbase = wid * qch
    pltpu.sync_copy(qpos_hbm.at[pl.ds(qbase * 3, qch * 3)], qpos_v)
    pltpu.sync_copy(psx_hbm, psx_v)
    pltpu.sync_copy(psy_hbm, psy_v)
    pltpu.sync_copy(psz_hbm, psz_v)
    pltpu.sync_copy(vsx_hbm, vsx_v)
    pltpu.sync_copy(vsy_hbm, vsy_v)
    pltpu.sync_copy(vsz_hbm, vsz_v)
    pltpu.sync_copy(starts_hbm, starts_v)

    lanes = lax.iota(jnp.int32, 16)
    big = jnp.full((16,), 1e9, jnp.float32)

    @pl.loop(0, qch, step=QB)
    def _qblock(qb):
        def _q(ql):
            qi = qb + ql
            qv = plsc.load_gather(qpos_v, [qi * 3 + jnp.minimum(lanes, 2)])
            qx = qv[0]
            qy = qv[1]
            qz = qv[2]
            def _ifloor(v):
                c = v.astype(jnp.int32)
                return jnp.where(c.astype(jnp.float32) > v, c - 1, c)

            cx = _ifloor(qx * INVH)
            cy = _ifloor(qy * INVH)
            cz = _ifloor(qz * INVH)
            qlv = jnp.full((16,), ql, jnp.int32)
            row6 = jnp.full((16,), 6, jnp.int32)
            for c in range(4):
                plsc.store_scatter(stage_v, [qlv, row6, lanes + 16 * c], big)
            z0 = jnp.maximum(cz - 1, 0)
            z1 = jnp.minimum(cz + 1, G - 1)

            def scan_run(cnt, dxc, dyc):
                gx = cx + dxc
                gy = cy + dyc
                ok = (gx >= 0) & (gx < G) & (gy >= 0) & (gy < G)
                col = (gx * G + gy) * G
                col = jnp.where(ok, col, 0)
                se_idx = jnp.where(lanes == 0, col + z0, col + z1 + 1)
                sev = plsc.load_gather(starts_v, [se_idx])
                s = jnp.where(ok, sev[0], 0)
                e = jnp.where(ok, sev[1], 0)
                nch = (e - s + 15) // 16

                def chunk(j, cnt):
                    base = s + j * 16
                    idxv = base + lanes
                    m_in = idxv < e
                    idxc = jnp.where(m_in, idxv, 0)
                    px = plsc.load_gather(psx_v, [idxc])
                    py = plsc.load_gather(psy_v, [idxc])
                    pz = plsc.load_gather(psz_v, [idxc])
                    dx = px - qx
                    dy = py - qy
                    dz = pz - qz
                    d2 = dx * dx + dy * dy + dz * dz
                    hit = m_in & (d2 <= R2)
                    hi = hit.astype(jnp.int32)
                    cpos = cnt + plsc.cumsum(hi) - 1
                    okm = hit & (cpos < KCAP)
                    vx = plsc.load_gather(vsx_v, [idxc])
                    vy = plsc.load_gather(vsy_v, [idxc])
                    vz = plsc.load_gather(vsz_v, [idxc])
                    r0 = jnp.zeros((16,), jnp.int32)
                    plsc.store_scatter(stage_v, [qlv, r0, cpos], dx, mask=okm)
                    plsc.store_scatter(stage_v, [qlv, r0 + 1, cpos], dy, mask=okm)
                    plsc.store_scatter(stage_v, [qlv, r0 + 2, cpos], dz, mask=okm)
                    plsc.store_scatter(stage_v, [qlv, r0 + 3, cpos], vx, mask=okm)
                    plsc.store_scatter(stage_v, [qlv, r0 + 4, cpos], vy, mask=okm)
                    plsc.store_scatter(stage_v, [qlv, r0 + 5, cpos], vz, mask=okm)
                    plsc.store_scatter(stage_v, [qlv, row6, cpos], d2, mask=okm)
                    return cpos[15] + 1

                return lax.fori_loop(0, nch, chunk, cnt)

            cnt = jnp.int32(0)
            for dxc in (-1, 0, 1):
                for dyc in (-1, 0, 1):
                    cnt = scan_run(cnt, dxc, dyc)

        @pl.loop(0, QB)
        def _qloop(ql):
            _q(ql)

        pltpu.sync_copy(stage_v, out_hbm.at[pl.ds(qbase + qb, QB)])


@functools.cache
def _sc_neighbors_kernel(qch):
  cp = pltpu.CompilerParams()
  if "needs_layout_passes" in pltpu.CompilerParams.__dataclass_fields__:
    cp = dataclasses.replace(cp, needs_layout_passes=False)
  return pl.kernel(
    functools.partial(_sc_body, qch),
    out_type=jax.ShapeDtypeStruct((NW * qch, 8, KCAP), jnp.float32),
    mesh=plsc.VectorSubcoreMesh(core_axis_name="c", subcore_axis_name="s",
                                num_cores=NC, num_subcores=NS),
    scratch_types=[
        pltpu.VMEM((NPAD,), jnp.float32),
        pltpu.VMEM((NPAD,), jnp.float32),
        pltpu.VMEM((NPAD,), jnp.float32),
        pltpu.VMEM((NPAD,), jnp.float32),
        pltpu.VMEM((NPAD,), jnp.float32),
        pltpu.VMEM((NPAD,), jnp.float32),
        pltpu.VMEM((NSTARTS,), jnp.int32),
        pltpu.VMEM((qch * 3,), jnp.float32),
        pltpu.VMEM((QB, 8, KCAP), jnp.float32),
    ],
    compiler_params=cp,
  )


def _tc_body(e_ref, w_ref, o_ref):
    e = e_ref[...]

    def plane(r):
        return jnp.transpose(e[:, r, :], (1, 0))

    d2 = plane(6)
    valid = d2 <= R2
    fv = valid.astype(jnp.float32)

    def san(r, fill):
        return jnp.where(valid, plane(r), np.float32(fill))

    dx = san(0, 1.0)
    dy = san(1, 0.0)
    dz = san(2, 0.0)
    vx = san(3, 0.0)
    vy = san(4, 0.0)
    vz = san(5, 0.0)

    scale = np.float32(2.0 / EXTENT)
    x = dx * scale
    y = dy * scale
    z = dz * scale

    sq = x * x + y * y + z * z
    norm = jnp.sqrt(jnp.maximum(sq, 1e-20))
    xy_sq = x * x + y * y
    cond = (5.0 * z * z / 4.0) <= xy_sq
    s_a = norm / jnp.sqrt(jnp.maximum(xy_sq, 1e-20))
    xa, ya, za = x * s_a, y * s_a, 1.5 * z
    s_b = jnp.sqrt(3.0 * norm / jnp.maximum(norm + jnp.abs(z), 1e-20))
    xb, yb, zb = x * s_b, y * s_b, jnp.sign(z) * norm
    xo = jnp.where(cond, xa, xb)
    yo = jnp.where(cond, ya, yb)
    zo = jnp.where(cond, za, zb)
    zero = sq < 1e-20
    x = jnp.where(zero, x, xo)
    y = jnp.where(zero, y, yo)
    z = jnp.where(zero, z, zo)

    sq_xy = x * x + y * y
    norm_xy = jnp.sqrt(jnp.maximum(sq_xy, 1e-20))
    cond = jnp.abs(y) <= jnp.abs(x)
    safe_x = jnp.where(jnp.abs(x) > 1e-10, x, np.float32(1e-10))
    safe_y = jnp.where(jnp.abs(y) > 1e-10, y, np.float32(1e-10))
    xa = jnp.sign(x) * norm_xy
    ya = jnp.sign(x) * _FOUR_OVER_PI * norm_xy * _atan(y / safe_x)
    yb = jnp.sign(y) * norm_xy
    xb = jnp.sign(y) * _FOUR_OVER_PI * norm_xy * _atan(x / safe_y)
    xo = jnp.where(cond, xa, xb)
    yo = jnp.where(cond, ya, yb)
    zero = sq_xy < 1e-20
    x = jnp.where(zero, x, xo)
    y = jnp.where(zero, y, yo)

    fx = (x * 0.5 + 0.5) * (FILTER - 1)
    fy = (y * 0.5 + 0.5) * (FILTER - 1)
    fz = (z * 0.5 + 0.5) * (FILTER - 1)

    def axw(f):
        f0 = jnp.floor(f)
        frac = f - f0
        i0 = jnp.clip(f0.astype(jnp.int32), 0, FILTER - 1)
        i1 = jnp.clip(f0.astype(jnp.int32) + 1, 0, FILTER - 1)
        return [(1.0 - frac) * (i0 == a).astype(jnp.float32)
                + frac * (i1 == a).astype(jnp.float32) for a in range(FILTER)]

    r_sqr = d2 / R2
    w_win = jnp.clip((1.0 - r_sqr) ** 3, 0.0, 1.0)
    imp = w_win * fv

    axl = axw(fx)
    ayl = axw(fy)
    azl = axw(fz)
    fim = [imp * vx, imp * vy, imp * vz]
    gp = [[axl[xx] * fim[i] for i in range(3)] for xx in range(4)]

    outs = [jnp.zeros((RB,), jnp.float32) for _ in range(3)]
    for a in range(4):
        for b in range(4):
            azay = azl[a] * ayl[b]
            for xx in range(4):
                for i in range(3):
                    s = jnp.sum(azay * gp[xx][i], axis=0)
                    widx = ((a * 4 + b) * 4 + xx) * 3 + i
                    for o in range(3):
                        outs[o] = outs[o] + s * w_ref[widx, o]
    o_ref[...] = jnp.stack(outs, axis=0)


@functools.cache
def _tc_conv(npart):
  return pl.pallas_call(
    _tc_body,
    grid=(npart // RB,),
    in_specs=[
        pl.BlockSpec((RB, 8, KCAP), lambda i: (i, 0, 0)),
        pl.BlockSpec(memory_space=pltpu.SMEM),
    ],
    out_specs=pl.BlockSpec((3, RB), lambda i: (0, i)),
    out_shape=jax.ShapeDtypeStruct((3, npart), jnp.float32),
  )


def kernel(pos, vel, W):
    pos = pos.astype(jnp.float32)
    vel = vel.astype(jnp.float32)
    cxyz = jnp.clip((pos * INVH).astype(jnp.int32), 0, G - 1)
    cid = (cxyz[:, 0] * G + cxyz[:, 1]) * G + cxyz[:, 2]
    sidx = jnp.argsort(cid)
    counts = jnp.zeros((NCELL,), jnp.int32).at[cid].add(1)
    starts = jnp.concatenate(
        [jnp.zeros((1,), jnp.int32), jnp.cumsum(counts, dtype=jnp.int32)])
    starts = jnp.pad(starts, (0, NSTARTS - (NCELL + 1)), constant_values=NP_)
    ps = pos[sidx]
    vs = vel[sidx]
    padn = NPAD - NP_

    def pad1(a):
        return jnp.pad(a, (0, padn))

    qpos = jnp.pad(pos, ((0, padn), (0, 0))).reshape(-1)
    planes = (pad1(ps[:, 0]), pad1(ps[:, 1]), pad1(ps[:, 2]),
              pad1(vs[:, 0]), pad1(vs[:, 1]), pad1(vs[:, 2]), starts)
    wmat = W.astype(jnp.float32).reshape(FILTER ** 3 * 3, 3)
    nsplit = 4
    part = NPAD // nsplit
    sc = _sc_neighbors_kernel(part // NW)
    tc = _tc_conv(part)
    outs = []
    for i in range(nsplit):
        e = sc(qpos[i * part * 3:(i + 1) * part * 3], *planes)
        outs.append(tc(e, wmat))
    out = jnp.concatenate(outs, axis=1)
    wc = 0.125 * jnp.sum(W.astype(jnp.float32)[1:3, 1:3, 1:3], axis=(0, 1, 2))
    return out[:, :NP_].T - vel @ wc


# speedup vs baseline: 1.1306x; 1.0328x over previous
import dataclasses
import functools
import math

import jax
import jax.numpy as jnp
import numpy as np
from jax import lax
from jax.experimental import pallas as pl
from jax.experimental.pallas import tpu as pltpu
from jax.experimental.pallas import tpu_sc as plsc

NP_ = 10000
KCAP = 64
FILTER = 4
EXTENT = np.float32(4 * 6 * 0.025)
RADIUS = np.float32(EXTENT / 2.0)
R2 = np.float32(RADIUS * RADIUS)
BOX = np.float32(3.3)
G = 11
INVH = np.float32(G / BOX)
NCELL = G * G * G

NC, NS = 2, 16
NW = NC * NS
QCH = 320
NPAD = NW * QCH
QB = 32
NSTARTS = 1336
RB = 512

_ATAN_C = (0.9999994161532382, -0.33330223018999183, 0.19951119254100916,
           -0.13933275185445243, 0.09709477935281635, -0.05688276598933052,
           0.02256821902867305, -0.004257820308274221)
_HALF_PI = np.float32(math.pi / 2)
_FOUR_OVER_PI = np.float32(4.0 / math.pi)


def _atan(t):
    at = jnp.abs(t)
    inv = at > 1.0
    z = jnp.where(inv, 1.0 / jnp.where(inv, t, 1.0), t)
    z2 = z * z
    p = jnp.float32(_ATAN_C[-1])
    for c in _ATAN_C[-2::-1]:
        p = p * z2 + np.float32(c)
    p = p * z
    return jnp.where(inv, jnp.sign(t) * _HALF_PI - p, p)


def _sc_body(qch, qpos_hbm, psx_hbm, psy_hbm, psz_hbm, vsx_hbm, vsy_hbm,
             vsz_hbm, starts_hbm, out_hbm,
             psx_v, psy_v, psz_v, vsx_v, vsy_v, vsz_v, starts_v,
             qpos_v, stage_v):
    wid = lax.axis_index("s") * NC + lax.axis_index("c")
    qbase = wid * qch
    pltpu.sync_copy(qpos_hbm.at[pl.ds(qbase * 3, qch * 3)], qpos_v)
    pltpu.sync_copy(psx_hbm, psx_v)
    pltpu.sync_copy(psy_hbm, psy_v)
    pltpu.sync_copy(psz_hbm, psz_v)
    pltpu.sync_copy(vsx_hbm, vsx_v)
    pltpu.sync_copy(vsy_hbm, vsy_v)
    pltpu.sync_copy(vsz_hbm, vsz_v)
    pltpu.sync_copy(starts_hbm, starts_v)

    lanes = lax.iota(jnp.int32, 16)
    big = jnp.full((16,), 1e9, jnp.float32)

    @pl.loop(0, qch, step=QB)
    def _qblock(qb):
        def _q(ql):
            qi = qb + ql
            qv = plsc.load_gather(qpos_v, [qi * 3 + jnp.minimum(lanes, 2)])
            qx = qv[0]
            qy = qv[1]
            qz = qv[2]
            def _ifloor(v):
                c = v.astype(jnp.int32)
                return jnp.where(c.astype(jnp.float32) > v, c - 1, c)

            cx = _ifloor(qx * INVH)
            cy = _ifloor(qy * INVH)
            cz = _ifloor(qz * INVH)
            qlv = jnp.full((16,), ql, jnp.int32)
            row6 = jnp.full((16,), 6, jnp.int32)
            for c in range(4):
                plsc.store_scatter(stage_v, [qlv, row6, lanes + 16 * c], big)
            z0 = jnp.maximum(cz - 1, 0)
            z1 = jnp.minimum(cz + 1, G - 1)

            def scan_run(cnt, dxc, dyc):
                gx = cx + dxc
                gy = cy + dyc
                ok = (gx >= 0) & (gx < G) & (gy >= 0) & (gy < G)
                col = (gx * G + gy) * G
                col = jnp.where(ok, col, 0)
                se_idx = jnp.where(lanes == 0, col + z0, col + z1 + 1)
                sev = plsc.load_gather(starts_v, [se_idx])
                s = jnp.where(ok, sev[0], 0)
                e = jnp.where(ok, sev[1], 0)
                nch = (e - s + 15) // 16

                def chunk(j, cnt):
                    base = s + j * 16
                    idxv = base + lanes
                    m_in = idxv < e
                    idxc = jnp.where(m_in, idxv, 0)
                    px = plsc.load_gather(psx_v, [idxc])
                    py = plsc.load_gather(psy_v, [idxc])
                    pz = plsc.load_gather(psz_v, [idxc])
                    dx = px - qx
                    dy = py - qy
                    dz = pz - qz
                    d2 = dx * dx + dy * dy + dz * dz
                    hit = m_in & (d2 <= R2)
                    hi = hit.astype(jnp.int32)
                    cpos = cnt + plsc.cumsum(hi) - 1
                    okm = hit & (cpos < KCAP)
                    vx = plsc.load_gather(vsx_v, [idxc])
                    vy = plsc.load_gather(vsy_v, [idxc])
                    vz = plsc.load_gather(vsz_v, [idxc])
                    r0 = jnp.zeros((16,), jnp.int32)
                    plsc.store_scatter(stage_v, [qlv, r0, cpos], dx, mask=okm)
                    plsc.store_scatter(stage_v, [qlv, r0 + 1, cpos], dy, mask=okm)
                    plsc.store_scatter(stage_v, [qlv, r0 + 2, cpos], dz, mask=okm)
                    plsc.store_scatter(stage_v, [qlv, r0 + 3, cpos], vx, mask=okm)
                    plsc.store_scatter(stage_v, [qlv, r0 + 4, cpos], vy, mask=okm)
                    plsc.store_scatter(stage_v, [qlv, r0 + 5, cpos], vz, mask=okm)
                    plsc.store_scatter(stage_v, [qlv, row6, cpos], d2, mask=okm)
                    return cpos[15] + 1

                return lax.fori_loop(0, nch, chunk, cnt)

            cnt = jnp.int32(0)
            for dxc in (-1, 0, 1):
                for dyc in (-1, 0, 1):
                    cnt = scan_run(cnt, dxc, dyc)

        @pl.loop(0, QB)
        def _qloop(ql):
            _q(ql)

        pltpu.sync_copy(stage_v, out_hbm.at[pl.ds(qbase + qb, QB)])


@functools.cache
def _sc_neighbors_kernel(qch):
  cp = pltpu.CompilerParams()
  if "needs_layout_passes" in pltpu.CompilerParams.__dataclass_fields__:
    cp = dataclasses.replace(cp, needs_layout_passes=False)
  return pl.kernel(
    functools.partial(_sc_body, qch),
    out_type=jax.ShapeDtypeStruct((NW * qch, 8, KCAP), jnp.float32),
    mesh=plsc.VectorSubcoreMesh(core_axis_name="c", subcore_axis_name="s",
                                num_cores=NC, num_subcores=NS),
    scratch_types=[
        pltpu.VMEM((NPAD,), jnp.float32),
        pltpu.VMEM((NPAD,), jnp.float32),
        pltpu.VMEM((NPAD,), jnp.float32),
        pltpu.VMEM((NPAD,), jnp.float32),
        pltpu.VMEM((NPAD,), jnp.float32),
        pltpu.VMEM((NPAD,), jnp.float32),
        pltpu.VMEM((NSTARTS,), jnp.int32),
        pltpu.VMEM((qch * 3,), jnp.float32),
        pltpu.VMEM((QB, 8, KCAP), jnp.float32),
    ],
    compiler_params=cp,
  )


def _tc_body(e_ref, w_ref, o_ref):
    e = e_ref[...]

    def plane(r):
        return jnp.transpose(e[:, r, :], (1, 0))

    d2 = plane(6)
    valid = d2 <= R2
    fv = valid.astype(jnp.float32)

    def san(r, fill):
        return jnp.where(valid, plane(r), np.float32(fill))

    dx = san(0, 1.0)
    dy = san(1, 0.0)
    dz = san(2, 0.0)
    vx = san(3, 0.0)
    vy = san(4, 0.0)
    vz = san(5, 0.0)

    scale = np.float32(2.0 / EXTENT)
    x = dx * scale
    y = dy * scale
    z = dz * scale

    sq = x * x + y * y + z * z
    norm = jnp.sqrt(jnp.maximum(sq, 1e-20))
    xy_sq = x * x + y * y
    cond = (5.0 * z * z / 4.0) <= xy_sq
    s_a = norm / jnp.sqrt(jnp.maximum(xy_sq, 1e-20))
    xa, ya, za = x * s_a, y * s_a, 1.5 * z
    s_b = jnp.sqrt(3.0 * norm / jnp.maximum(norm + jnp.abs(z), 1e-20))
    xb, yb, zb = x * s_b, y * s_b, jnp.sign(z) * norm
    xo = jnp.where(cond, xa, xb)
    yo = jnp.where(cond, ya, yb)
    zo = jnp.where(cond, za, zb)
    zero = sq < 1e-20
    x = jnp.where(zero, x, xo)
    y = jnp.where(zero, y, yo)
    z = jnp.where(zero, z, zo)

    sq_xy = x * x + y * y
    norm_xy = jnp.sqrt(jnp.maximum(sq_xy, 1e-20))
    cond = jnp.abs(y) <= jnp.abs(x)
    safe_x = jnp.where(jnp.abs(x) > 1e-10, x, np.float32(1e-10))
    safe_y = jnp.where(jnp.abs(y) > 1e-10, y, np.float32(1e-10))
    xa = jnp.sign(x) * norm_xy
    ya = jnp.sign(x) * _FOUR_OVER_PI * norm_xy * _atan(y / safe_x)
    yb = jnp.sign(y) * norm_xy
    xb = jnp.sign(y) * _FOUR_OVER_PI * norm_xy * _atan(x / safe_y)
    xo = jnp.where(cond, xa, xb)
    yo = jnp.where(cond, ya, yb)
    zero = sq_xy < 1e-20
    x = jnp.where(zero, x, xo)
    y = jnp.where(zero, y, yo)

    fx = (x * 0.5 + 0.5) * (FILTER - 1)
    fy = (y * 0.5 + 0.5) * (FILTER - 1)
    fz = (z * 0.5 + 0.5) * (FILTER - 1)

    def axw(f):
        f0 = jnp.floor(f)
        frac = f - f0
        i0 = jnp.clip(f0.astype(jnp.int32), 0, FILTER - 1)
        i1 = jnp.clip(f0.astype(jnp.int32) + 1, 0, FILTER - 1)
        return [(1.0 - frac) * (i0 == a).astype(jnp.float32)
                + frac * (i1 == a).astype(jnp.float32) for a in range(FILTER)]

    r_sqr = d2 / R2
    w_win = jnp.clip((1.0 - r_sqr) ** 3, 0.0, 1.0)
    imp = w_win * fv

    axl = axw(fx)
    ayl = axw(fy)
    azl = axw(fz)
    fim = [imp * vx, imp * vy, imp * vz]
    gp = [[axl[xx] * fim[i] for i in range(3)] for xx in range(4)]

    outs = [jnp.zeros((RB,), jnp.float32) for _ in range(3)]
    for a in range(4):
        for b in range(4):
            azay = azl[a] * ayl[b]
            for xx in range(4):
                for i in range(3):
                    s = jnp.sum(azay * gp[xx][i], axis=0)
                    widx = ((a * 4 + b) * 4 + xx) * 3 + i
                    for o in range(3):
                        outs[o] = outs[o] + s * w_ref[widx, o]
    o_ref[...] = jnp.stack(outs, axis=0)


@functools.cache
def _tc_conv(npart):
  return pl.pallas_call(
    _tc_body,
    grid=(npart // RB,),
    in_specs=[
        pl.BlockSpec((RB, 8, KCAP), lambda i: (i, 0, 0)),
        pl.BlockSpec(memory_space=pltpu.SMEM),
    ],
    out_specs=pl.BlockSpec((3, RB), lambda i: (0, i)),
    out_shape=jax.ShapeDtypeStruct((3, npart), jnp.float32),
  )


def kernel(pos, vel, W):
    pos = pos.astype(jnp.float32)
    vel = vel.astype(jnp.float32)
    cxyz = jnp.clip((pos * INVH).astype(jnp.int32), 0, G - 1)
    cid = (cxyz[:, 0] * G + cxyz[:, 1]) * G + cxyz[:, 2]
    sidx = jnp.argsort(cid)
    counts = jnp.zeros((NCELL,), jnp.int32).at[cid].add(1)
    starts = jnp.concatenate(
        [jnp.zeros((1,), jnp.int32), jnp.cumsum(counts, dtype=jnp.int32)])
    starts = jnp.pad(starts, (0, NSTARTS - (NCELL + 1)), constant_values=NP_)
    ps = pos[sidx]
    vs = vel[sidx]
    padn = NPAD - NP_

    def pad1(a):
        return jnp.pad(a, (0, padn))

    qpos = jnp.pad(pos, ((0, padn), (0, 0))).reshape(-1)
    planes = (pad1(ps[:, 0]), pad1(ps[:, 1]), pad1(ps[:, 2]),
              pad1(vs[:, 0]), pad1(vs[:, 1]), pad1(vs[:, 2]), starts)
    wmat = W.astype(jnp.float32).reshape(FILTER ** 3 * 3, 3)
    nsplit = 2
    part = NPAD // nsplit
    sc = _sc_neighbors_kernel(part // NW)
    tc = _tc_conv(part)
    outs = []
    for i in range(nsplit):
        e = sc(qpos[i * part * 3:(i + 1) * part * 3], *planes)
        outs.append(tc(e, wmat))
    out = jnp.concatenate(outs, axis=1)
    wc = 0.125 * jnp.sum(W.astype(jnp.float32)[1:3, 1:3, 1:3], axis=(0, 1, 2))
    return out[:, :NP_].T - vel @ wc
